# baseline (speedup 1.0000x reference)
import numpy as np
import jax
import jax.numpy as jnp
from jax.experimental import pallas as pl

H_UNITS_K = 128


def _pos_enc_table_np(seq, h_units):
    pos = np.arange(seq).astype(np.float64)[:, None]
    i = np.arange(h_units).astype(np.float64)[None, :]
    enc = pos / np.power(10000.0, 2.0 * i / float(h_units))
    enc = enc.astype(np.float32)
    enc[:, 0::2] = np.sin(enc[:, 0::2])
    enc[:, 1::2] = np.cos(enc[:, 1::2])
    return enc


def kernel(inputs):
    bs, seq = inputs.shape
    h = H_UNITS_K
    table = jnp.asarray(_pos_enc_table_np(seq, h))

    BB = 64
    assert bs % BB == 0

    def body(tab_ref, out_ref):
        out_ref[...] = jnp.broadcast_to(tab_ref[...][None], (BB, seq, h))

    out = pl.pallas_call(
        body,
        grid=(bs // BB,),
        in_specs=[pl.BlockSpec((seq, h), lambda i: (0, 0))],
        out_specs=pl.BlockSpec((BB, seq, h), lambda i: (i, 0, 0)),
        out_shape=jax.ShapeDtypeStruct((bs, seq, h), jnp.float32),
    )(table)
    return out
